# baseline (speedup 1.0000x reference)
import functools

import jax
import jax.numpy as jnp
from jax import lax
from jax.experimental import pallas as pl
from jax.experimental.pallas import tpu as pltpu
from jax.experimental.pallas import tpu_sc as plsc

EMBED_DIM = 64
NUM_CORES = 2
NUM_SUBCORES = 16
NUM_WORKERS = NUM_CORES * NUM_SUBCORES
IDX_CHUNK = 128
LANES = 16


def _newton_sqrt(x):
    i = plsc.bitcast(x, jnp.int32)
    y = plsc.bitcast(jnp.int32(0x5F3759DF) - (i >> 1), jnp.float32)
    y = y * (1.5 - 0.5 * x * y * y)
    y = y * (1.5 - 0.5 * x * y * y)
    y = y * (1.5 - 0.5 * x * y * y)
    return jnp.where(x > 0.0, x * y, 0.0)


def _make_sc_kernel(batch):
    bpw = batch // NUM_WORKERS
    n_chunks = bpw // IDX_CHUNK
    mesh = plsc.VectorSubcoreMesh(core_axis_name="c", subcore_axis_name="s")

    @functools.partial(
        pl.kernel,
        mesh=mesh,
        compiler_params=pltpu.CompilerParams(
            needs_layout_passes=False, use_tc_tiling_on_sc=False),
        out_type=jax.ShapeDtypeStruct((batch,), jnp.float32),
        scratch_types=[
            pltpu.VMEM((n_chunks, IDX_CHUNK), jnp.int32),
            pltpu.VMEM((n_chunks, IDX_CHUNK), jnp.int32),
            pltpu.VMEM((n_chunks, IDX_CHUNK), jnp.int32),
            pltpu.VMEM((bpw, EMBED_DIM), jnp.float32),
            pltpu.VMEM((bpw, EMBED_DIM), jnp.float32),
            pltpu.VMEM((bpw, EMBED_DIM), jnp.float32),
            pltpu.VMEM((bpw,), jnp.float32),
            pltpu.SemaphoreType.DMA,
        ],
    )
    def sc_kernel(ent_hbm, rel_hbm, h_hbm, r_hbm, t_hbm, out_hbm,
                  hidx, ridx, tidx, hrow, rrow, trow, outv, sem):
        wid = lax.axis_index("s") * NUM_CORES + lax.axis_index("c")
        base = wid * bpw

        pltpu.sync_copy(h_hbm.at[wid], hidx)
        pltpu.sync_copy(r_hbm.at[wid], ridx)
        pltpu.sync_copy(t_hbm.at[wid], tidx)

        copies = []
        for j in range(n_chunks):
            dst = pl.ds(j * IDX_CHUNK, IDX_CHUNK)
            copies.append(pltpu.async_copy(ent_hbm.at[hidx.at[j]], hrow.at[dst], sem))
            copies.append(pltpu.async_copy(rel_hbm.at[ridx.at[j]], rrow.at[dst], sem))
            copies.append(pltpu.async_copy(ent_hbm.at[tidx.at[j]], trow.at[dst], sem))
        for c in copies:
            c.wait()

        lane = lax.iota(jnp.int32, LANES)

        def body(g, carry):
            vec = jnp.zeros((LANES,), jnp.float32)
            for u in range(LANES):
                b = g * LANES + u
                acc = jnp.zeros((LANES,), jnp.float32)
                for c in range(EMBED_DIM // LANES):
                    sl = pl.ds(c * LANES, LANES)
                    diff = hrow[b, sl] + rrow[b, sl] - trow[b, sl]
                    acc = acc + diff * diff
                vec = jnp.where(lane == u, jnp.sum(acc), vec)
            outv[pl.ds(g * LANES, LANES)] = -_newton_sqrt(vec)
            return carry

        lax.fori_loop(0, bpw // LANES, body, 0)

        pltpu.sync_copy(outv, out_hbm.at[pl.ds(base, bpw)])

    return sc_kernel


def kernel(entity_table, relation_table, h, r, t):
    batch = h.shape[0]
    bpw = batch // NUM_WORKERS
    n_chunks = bpw // IDX_CHUNK
    shape3 = (NUM_WORKERS, n_chunks, IDX_CHUNK)
    h3 = h.astype(jnp.int32).reshape(shape3)
    r3 = r.astype(jnp.int32).reshape(shape3)
    t3 = t.astype(jnp.int32).reshape(shape3)
    return _make_sc_kernel(batch)(entity_table, relation_table, h3, r3, t3)
